# Initial kernel scaffold; baseline (speedup 1.0000x reference)
import jax
import jax.numpy as jnp
from jax.experimental import pallas as pl


def kernel(x, bd_pred, Wc, bc, Wlin, blin, Wlins, blins, Wl1, bl1, Wr1, Wl2, bl2, Wr2, Wl3, bl3, Wr3, ln1_g, ln1_b, ln2_g, ln2_b, mp1_W, mp1_b, mp2_W, mp2_b, edge_index):
    raise NotImplementedError("write your pallas kernel here")


# speedup vs baseline: 3.2805x; 3.2805x over previous
import functools

import numpy as np
import jax
import jax.numpy as jnp
from jax import lax
from jax.experimental import pallas as pl
from jax.experimental.pallas import tpu as pltpu
from jax.experimental.pallas import tpu_sc as plsc

N = 50000
E = 800000
HID = 64
F2 = 32
NB = 1000
NBLK = N // NB

B = 128
T = 4
SB = T * B
NSC = 100
EPT = SB * NSC
EPAD = 16 * EPT
NA = 50176
DUMMY = NA - 1
RPT = NA // 16
ZR = 196
OCH = 392

_f32 = jnp.float32


def _sc_agg_body(ht0, ht1, src2, dst2, out0, out1, acc, zbuf, srcv, dstv,
                 gath, sem, sem2):
    c = lax.axis_index("c")
    s = lax.axis_index("s")

    def _zb(i, car):
        zbuf[i, pl.ds(0, 16)] = jnp.zeros((16,), _f32)
        zbuf[i, pl.ds(16, 16)] = jnp.zeros((16,), _f32)
        return car
    lax.fori_loop(0, ZR, _zb, 0)

    def _zc(k, car):
        pltpu.sync_copy(zbuf, acc.at[pl.ds(s * RPT + k * ZR, ZR)])
        return car
    lax.fori_loop(0, RPT // ZR, _zc, 0)
    plsc.subcore_barrier()

    def _run(tab):
        def _chunk(j, car):
            r0 = s * (EPT // B) + j * T
            pltpu.sync_copy(src2.at[pl.ds(r0, T)], srcv)
            pltpu.sync_copy(dst2.at[pl.ds(r0, T)], dstv)
            hs = [pltpu.async_copy(tab.at[srcv.at[t]],
                                   gath.at[pl.ds(t * B, B)], sem)
                  for t in range(T)]
            for h in hs:
                h.wait()
            hs2 = [pltpu.async_copy(gath.at[pl.ds(t * B, B)],
                                    acc.at[dstv.at[t]], sem2, add=True)
                   for t in range(T)]
            for h in hs2:
                h.wait()
            return car
        lax.fori_loop(0, NSC, _chunk, 0)

    @pl.when(c == 0)
    def _():
        _run(ht0)

    @pl.when(c == 1)
    def _():
        _run(ht1)

    plsc.subcore_barrier()

    def _out(dst):
        def _cp(k, car):
            r = s * RPT + k * OCH
            pltpu.sync_copy(acc.at[pl.ds(r, OCH)], gath.at[pl.ds(0, OCH)])
            pltpu.sync_copy(gath.at[pl.ds(0, OCH)], dst.at[pl.ds(r, OCH)])
            return car
        lax.fori_loop(0, RPT // OCH, _cp, 0)

    @pl.when(c == 0)
    def _():
        _out(out0)

    @pl.when(c == 1)
    def _():
        _out(out1)


def _sc_deg_body(src2, dst2, outd, outs, accd, accs, zbufd, srcv, dstv,
                 slcidx, ones, sem):
    c = lax.axis_index("c")
    s = lax.axis_index("s")

    @pl.when(c == 0)
    def _():
        def _zb(i, car):
            zbufd[pl.ds(i * 16, 16)] = jnp.zeros((16,), _f32)
            return car
        lax.fori_loop(0, (NA // 16) // 16, _zb, 0)
        for t in range(8):
            ones[pl.ds(t * 16, 16)] = jnp.ones((16,), _f32)
        pltpu.sync_copy(zbufd, accd.at[pl.ds(s * (NA // 16), NA // 16)])
        pltpu.sync_copy(zbufd, accs.at[pl.ds(s * (NA // 16), NA // 16)])
        plsc.subcore_barrier()

        def _chunk(j, car):
            r0 = s * (EPT // B) + j * T
            pltpu.sync_copy(src2.at[pl.ds(r0, T)], srcv)
            pltpu.sync_copy(dst2.at[pl.ds(r0, T)], dstv)
            for t in range(T):
                def _cmp(i, car2):
                    sv = srcv[t, pl.ds(i * 16, 16)]
                    dv = dstv[t, pl.ds(i * 16, 16)]
                    slcidx[t, pl.ds(i * 16, 16)] = jnp.where(
                        sv == dv, dv, jnp.full((16,), DUMMY, jnp.int32))
                    return car2
                lax.fori_loop(0, B // 16, _cmp, 0)
            hs = []
            for t in range(T):
                hs.append(pltpu.async_copy(ones, accd.at[dstv.at[t]], sem,
                                           add=True))
                hs.append(pltpu.async_copy(ones, accs.at[slcidx.at[t]], sem,
                                           add=True))
            for h in hs:
                h.wait()
            return car
        lax.fori_loop(0, NSC, _chunk, 0)
        plsc.subcore_barrier()
        r = s * (NA // 16)
        pltpu.sync_copy(accd.at[pl.ds(r, NA // 16)], zbufd)
        pltpu.sync_copy(zbufd, outd.at[pl.ds(r, NA // 16)])
        pltpu.sync_copy(accs.at[pl.ds(r, NA // 16)], zbufd)
        pltpu.sync_copy(zbufd, outs.at[pl.ds(r, NA // 16)])


_SC_MESH = plsc.VectorSubcoreMesh(core_axis_name="c", subcore_axis_name="s")

_sc_agg = pl.kernel(
    _sc_agg_body,
    out_type=[jax.ShapeDtypeStruct((NA, F2), _f32),
              jax.ShapeDtypeStruct((NA, F2), _f32)],
    mesh=_SC_MESH,
    compiler_params=pltpu.CompilerParams(use_tc_tiling_on_sc=False),
    scratch_types=[
        pltpu.VMEM_SHARED((NA, F2), _f32),
        pltpu.VMEM((ZR, F2), _f32),
        pltpu.VMEM((T, B), jnp.int32),
        pltpu.VMEM((T, B), jnp.int32),
        pltpu.VMEM((SB, F2), _f32),
        pltpu.SemaphoreType.DMA,
        pltpu.SemaphoreType.DMA,
    ],
)

_sc_deg = pl.kernel(
    _sc_deg_body,
    out_type=[jax.ShapeDtypeStruct((NA,), _f32),
              jax.ShapeDtypeStruct((NA,), _f32)],
    mesh=_SC_MESH,
    compiler_params=pltpu.CompilerParams(use_tc_tiling_on_sc=False),
    scratch_types=[
        pltpu.VMEM_SHARED((NA,), _f32),
        pltpu.VMEM_SHARED((NA,), _f32),
        pltpu.VMEM((NA // 16,), _f32),
        pltpu.VMEM((T, B), jnp.int32),
        pltpu.VMEM((T, B), jnp.int32),
        pltpu.VMEM((T, B), jnp.int32),
        pltpu.VMEM((B,), _f32),
        pltpu.SemaphoreType.DMA,
    ],
)


def _dot(a, b):
    return jnp.dot(a, b, preferred_element_type=_f32)


def _tc_a_body(x2, bd, M, bc, wst, bs, wnt, bn, sx, h0, h1):
    y = _dot(x2[...], M[...]) + bc[...]
    r = jax.nn.relu(jnp.concatenate([y, bd[...]], axis=1))
    sx[...] = _dot(r, wst[...]) + bs[...]
    xnb = _dot(r, wnt[...]) + bn[...]
    h0[...] = xnb[:, :F2]
    h1[...] = xnb[:, F2:]


def _tc_1_body(sx, a0, a1, x0, x1, dg, sl, wrt, bl, h0, h1, hr, dinv):
    xnb = jnp.concatenate([x0[...], x1[...]], axis=1)
    h = sx[...] + jnp.concatenate([a0[...], a1[...]], axis=1) - sl[...] * xnb
    h0[...] = h[:, :F2]
    h1[...] = h[:, F2:]
    hr[...] = _dot(h, wrt[...]) + bl[...]
    dinv[...] = 1.0 / jnp.maximum(dg[...], 1.0)


def _tc_mid_body(a0, a1, dinv, hri, wlt, g, b, wrt, bl, h0, h1, hro):
    u = _dot(jnp.concatenate([a0[...], a1[...]], axis=1) * dinv[...],
             wlt[...]) + hri[...]
    v = jax.nn.relu(u)
    mu = jnp.mean(v, axis=1, keepdims=True)
    var = jnp.mean((v - mu) ** 2, axis=1, keepdims=True)
    h = (v - mu) * lax.rsqrt(var + 1e-5) * g[...] + b[...]
    h0[...] = h[:, :F2]
    h1[...] = h[:, F2:]
    hro[...] = _dot(h, wrt[...]) + bl[...]


def _tc_4_body(a0, a1, dinv, hri, wlt, w1, b1, w2, b2, emb, lsm):
    e = _dot(jnp.concatenate([a0[...], a1[...]], axis=1) * dinv[...],
             wlt[...]) + hri[...]
    emb[...] = e
    p = _dot(jax.nn.relu(e), w1[...]) + b1[...]
    q = _dot(p, w2[...]) + b2[...]
    m = jnp.max(q, axis=1, keepdims=True)
    lsm[...] = q - (jnp.log(jnp.sum(jnp.exp(q - m), axis=1, keepdims=True))
                    + m)


def _rows(shape):
    return pl.BlockSpec((NB,) + shape[1:], lambda i: (i,) + (0,) * (len(shape) - 1))


def _full(shape):
    return pl.BlockSpec(shape, lambda i: (0,) * len(shape))


def _tc_call(body, ins, n_out, out_shapes):
    specs = [_rows(a.shape) if a.shape[0] == N else _full(a.shape) for a in ins]
    return pl.pallas_call(
        body,
        grid=(NBLK,),
        in_specs=specs,
        out_specs=[_rows(s) for s in out_shapes],
        out_shape=[jax.ShapeDtypeStruct(s, _f32) for s in out_shapes],
    )(*ins)


def _build_conv_matrix(Wc, bc):
    o, c, di, dj, p, q = np.meshgrid(np.arange(3), np.arange(3), np.arange(3),
                                     np.arange(3), np.arange(6), np.arange(6),
                                     indexing="ij")
    k = (c * 64 + (p + di) * 8 + (q + dj)).ravel()
    m = (o * 36 + p * 6 + q).ravel()
    w = Wc[o.ravel(), c.ravel(), di.ravel(), dj.ravel()]
    M = jnp.zeros((192, 108), _f32).at[k, m].add(w)
    return M, jnp.repeat(bc, 36)[None, :]


def kernel(x, bd_pred, Wc, bc, Wlin, blin, Wlins, blins, Wl1, bl1, Wr1, Wl2,
           bl2, Wr2, Wl3, bl3, Wr3, ln1_g, ln1_b, ln2_g, ln2_b, mp1_W, mp1_b,
           mp2_W, mp2_b, edge_index):
    M, bcvec = _build_conv_matrix(Wc, bc)
    x2 = x.reshape(N, 192)
    pad = EPAD - E
    src2 = jnp.concatenate(
        [edge_index[0], jnp.zeros((pad,), jnp.int32)]).reshape(EPAD // B, B)
    dst2 = jnp.concatenate(
        [edge_index[1], jnp.full((pad,), DUMMY, jnp.int32)]).reshape(EPAD // B, B)

    row = lambda v: v[None, :]

    outd, outs = _sc_deg(src2, dst2)
    deg = outd[:N].reshape(N, 1)
    slc = outs[:N].reshape(N, 1)

    sx, h0, h1 = _tc_call(
        _tc_a_body,
        [x2, bd_pred, M, bcvec, Wlins.T, row(blins), Wlin.T, row(blin)],
        3, [(N, HID), (N, F2), (N, F2)])

    a0, a1 = _sc_agg(h0, h1, src2, dst2)
    a0, a1 = a0[:N], a1[:N]
    h0, h1, hr, dinv = _tc_call(
        _tc_1_body,
        [sx, a0, a1, h0, h1, deg, slc, Wr1.T, row(bl1)],
        4, [(N, F2), (N, F2), (N, HID), (N, 1)])

    a0, a1 = _sc_agg(h0, h1, src2, dst2)
    a0, a1 = a0[:N], a1[:N]
    h0, h1, hr = _tc_call(
        _tc_mid_body,
        [a0, a1, dinv, hr, Wl1.T, row(ln1_g), row(ln1_b), Wr2.T, row(bl2)],
        3, [(N, F2), (N, F2), (N, HID)])

    a0, a1 = _sc_agg(h0, h1, src2, dst2)
    a0, a1 = a0[:N], a1[:N]
    h0, h1, hr = _tc_call(
        _tc_mid_body,
        [a0, a1, dinv, hr, Wl2.T, row(ln2_g), row(ln2_b), Wr3.T, row(bl3)],
        3, [(N, F2), (N, F2), (N, HID)])

    a0, a1 = _sc_agg(h0, h1, src2, dst2)
    a0, a1 = a0[:N], a1[:N]
    emb, lsm = _tc_call(
        _tc_4_body,
        [a0, a1, dinv, hr, Wl3.T, mp1_W.T, row(mp1_b), mp2_W.T, row(mp2_b)],
        2, [(N, HID), (N, 8)])

    return emb, lsm


# speedup vs baseline: 4.3980x; 1.3406x over previous
import functools

import numpy as np
import jax
import jax.numpy as jnp
from jax import lax
from jax.experimental import pallas as pl
from jax.experimental.pallas import tpu as pltpu
from jax.experimental.pallas import tpu_sc as plsc

N = 50000
E = 800000
HID = 64
F2 = 32
NB = 1000
NBLK = N // NB

B = 128
T = 2
RT = 400
NSC = RT // T
EROWS = 6528
EPAD = EROWS * B
DT = 4
DROWS = 204
NA = 50176
DUMMY = NA - 1
RPT = NA // 16
ZR = 196
OCH = 224

_f32 = jnp.float32


def _sc_agg_body(ht0, ht1, src2, dst2, out0, out1, acc, zbuf, srcA, dstA,
                 srcB, dstB, gA, gB, semG, semS):
    c = lax.axis_index("c")
    s = lax.axis_index("s")

    def _zb(i, car):
        zbuf[i, pl.ds(0, 16)] = jnp.zeros((16,), _f32)
        zbuf[i, pl.ds(16, 16)] = jnp.zeros((16,), _f32)
        return car
    lax.fori_loop(0, ZR, _zb, 0)

    def _zc(k, car):
        pltpu.sync_copy(zbuf, acc.at[pl.ds(s * RPT + k * ZR, ZR)])
        return car
    lax.fori_loop(0, RPT // ZR, _zc, 0)
    plsc.subcore_barrier()

    base = s * RT

    def _run(tab):
        def _ld(r, sv, dv):
            pltpu.sync_copy(src2.at[pl.ds(r, T)], sv)
            pltpu.sync_copy(dst2.at[pl.ds(r, T)], dv)

        def _gfire(sv, gbuf):
            for t in range(T):
                pltpu.async_copy(tab.at[sv.at[t]], gbuf.at[pl.ds(t * B, B)],
                                 semG)

        def _gwait(sv, gbuf):
            for t in range(T):
                pltpu.make_async_copy(tab.at[sv.at[t]],
                                      gbuf.at[pl.ds(t * B, B)], semG).wait()

        def _sfire(dv, gbuf):
            for t in range(T):
                pltpu.async_copy(gbuf.at[pl.ds(t * B, B)], acc.at[dv.at[t]],
                                 semS, add=True)

        def _swait(dv, gbuf):
            for t in range(T):
                pltpu.make_async_copy(gbuf.at[pl.ds(t * B, B)],
                                      acc.at[dv.at[t]], semS).wait()

        _ld(base, srcA, dstA)
        _gfire(srcA, gA)

        def _pair(k, car):
            @pl.when(k > 0)
            def _():
                _swait(dstB, gB)
            _ld(base + (2 * k + 1) * T, srcB, dstB)
            _gwait(srcA, gA)
            _gfire(srcB, gB)
            _sfire(dstA, gA)
            _swait(dstA, gA)
            _ld(base + (2 * k + 2) * T, srcA, dstA)
            _gwait(srcB, gB)
            _gfire(srcA, gA)
            _sfire(dstB, gB)
            return car
        lax.fori_loop(0, NSC // 2, _pair, 0)
        _gwait(srcA, gA)
        _swait(dstB, gB)

    @pl.when(c == 0)
    def _():
        _run(ht0)

    @pl.when(c == 1)
    def _():
        _run(ht1)

    plsc.subcore_barrier()

    def _out(dst):
        def _cp(k, car):
            r = s * RPT + k * OCH
            pltpu.sync_copy(acc.at[pl.ds(r, OCH)], gA.at[pl.ds(0, OCH)])
            pltpu.sync_copy(gA.at[pl.ds(0, OCH)], dst.at[pl.ds(r, OCH)])
            return car
        lax.fori_loop(0, RPT // OCH, _cp, 0)

    @pl.when(c == 0)
    def _():
        _out(out0)

    @pl.when(c == 1)
    def _():
        _out(out1)


def _sc_deg_body(src2, dst2, outd, dst0, accd, zbufd, srcv, dstv, d0v,
                 ones, sem):
    c = lax.axis_index("c")
    s = lax.axis_index("s")

    def _zb(i, car):
        zbufd[pl.ds(i * 16, 16)] = jnp.zeros((16,), _f32)
        return car
    lax.fori_loop(0, RPT // 16, _zb, 0)
    for t in range(8):
        ones[pl.ds(t * 16, 16)] = jnp.ones((16,), _f32)
    pltpu.sync_copy(zbufd, accd.at[pl.ds(s * RPT, RPT)])
    plsc.subcore_barrier()

    rbase = c * (EROWS // 2) + s * DROWS

    def _chunk(j, car):
        r0 = rbase + j * DT
        pltpu.sync_copy(src2.at[pl.ds(r0, DT)], srcv)
        pltpu.sync_copy(dst2.at[pl.ds(r0, DT)], dstv)
        for t in range(DT):
            def _cmp(i, car2):
                sv = srcv[t, pl.ds(i * 16, 16)]
                dv = dstv[t, pl.ds(i * 16, 16)]
                d0v[t, pl.ds(i * 16, 16)] = jnp.where(
                    sv == dv, jnp.full((16,), DUMMY, jnp.int32), dv)
                return car2
            lax.fori_loop(0, B // 16, _cmp, 0)
        for t in range(DT):
            pltpu.async_copy(ones, accd.at[dstv.at[t]], sem, add=True)
        pltpu.sync_copy(d0v, dst0.at[pl.ds(r0, DT)])
        for t in range(DT):
            pltpu.make_async_copy(ones, accd.at[dstv.at[t]], sem).wait()
        return car
    lax.fori_loop(0, DROWS // DT, _chunk, 0)
    plsc.subcore_barrier()
    pltpu.sync_copy(accd.at[pl.ds(s * RPT, RPT)], zbufd)
    pltpu.sync_copy(zbufd, outd.at[pl.ds(c * NA + s * RPT, RPT)])


_SC_MESH = plsc.VectorSubcoreMesh(core_axis_name="c", subcore_axis_name="s")

_sc_agg = pl.kernel(
    _sc_agg_body,
    out_type=[jax.ShapeDtypeStruct((NA, F2), _f32),
              jax.ShapeDtypeStruct((NA, F2), _f32)],
    mesh=_SC_MESH,
    compiler_params=pltpu.CompilerParams(use_tc_tiling_on_sc=False),
    scratch_types=[
        pltpu.VMEM_SHARED((NA, F2), _f32),
        pltpu.VMEM((ZR, F2), _f32),
        pltpu.VMEM((T, B), jnp.int32),
        pltpu.VMEM((T, B), jnp.int32),
        pltpu.VMEM((T, B), jnp.int32),
        pltpu.VMEM((T, B), jnp.int32),
        pltpu.VMEM((T * B, F2), _f32),
        pltpu.VMEM((T * B, F2), _f32),
        pltpu.SemaphoreType.DMA,
        pltpu.SemaphoreType.DMA,
    ],
)

_sc_deg = pl.kernel(
    _sc_deg_body,
    out_type=[jax.ShapeDtypeStruct((2 * NA,), _f32),
              jax.ShapeDtypeStruct((EROWS, B), jnp.int32)],
    mesh=_SC_MESH,
    compiler_params=pltpu.CompilerParams(use_tc_tiling_on_sc=False),
    scratch_types=[
        pltpu.VMEM_SHARED((NA,), _f32),
        pltpu.VMEM((RPT,), _f32),
        pltpu.VMEM((DT, B), jnp.int32),
        pltpu.VMEM((DT, B), jnp.int32),
        pltpu.VMEM((DT, B), jnp.int32),
        pltpu.VMEM((B,), _f32),
        pltpu.SemaphoreType.DMA,
    ],
)


def _dot(a, b):
    return jnp.dot(a, b, preferred_element_type=_f32)


def _tc_a_body(x2, bd, M, bc, wst, bs, wnt, bn, sx, h0, h1):
    y = _dot(x2[...], M[...]) + bc[...]
    r = jax.nn.relu(jnp.concatenate([y, bd[...]], axis=1))
    sx[...] = _dot(r, wst[...]) + bs[...]
    xnb = _dot(r, wnt[...]) + bn[...]
    h0[...] = xnb[:, :F2]
    h1[...] = xnb[:, F2:]


def _tc_1_body(sx, a0, a1, dg, wrt, bl, h0, h1, hr, dinv):
    h = sx[...] + jnp.concatenate([a0[...], a1[...]], axis=1)
    h0[...] = h[:, :F2]
    h1[...] = h[:, F2:]
    hr[...] = _dot(h, wrt[...]) + bl[...]
    dinv[...] = 1.0 / jnp.maximum(dg[...], 1.0)


def _tc_mid_body(a0, a1, dinv, hri, wlt, g, b, wrt, bl, h0, h1, hro):
    u = _dot(jnp.concatenate([a0[...], a1[...]], axis=1) * dinv[...],
             wlt[...]) + hri[...]
    v = jax.nn.relu(u)
    mu = jnp.mean(v, axis=1, keepdims=True)
    var = jnp.mean((v - mu) ** 2, axis=1, keepdims=True)
    h = (v - mu) * lax.rsqrt(var + 1e-5) * g[...] + b[...]
    h0[...] = h[:, :F2]
    h1[...] = h[:, F2:]
    hro[...] = _dot(h, wrt[...]) + bl[...]


def _tc_4_body(a0, a1, dinv, hri, wlt, w1, b1, w2, b2, emb, lsm):
    e = _dot(jnp.concatenate([a0[...], a1[...]], axis=1) * dinv[...],
             wlt[...]) + hri[...]
    emb[...] = e
    p = _dot(jax.nn.relu(e), w1[...]) + b1[...]
    q = _dot(p, w2[...]) + b2[...]
    m = jnp.max(q, axis=1, keepdims=True)
    lsm[...] = q - (jnp.log(jnp.sum(jnp.exp(q - m), axis=1, keepdims=True))
                    + m)


def _rows(shape):
    return pl.BlockSpec((NB,) + shape[1:], lambda i: (i,) + (0,) * (len(shape) - 1))


def _full(shape):
    return pl.BlockSpec(shape, lambda i: (0,) * len(shape))


def _tc_call(body, ins, n_out, out_shapes):
    specs = [_rows(a.shape) if a.shape[0] == N else _full(a.shape) for a in ins]
    return pl.pallas_call(
        body,
        grid=(NBLK,),
        in_specs=specs,
        out_specs=[_rows(s) for s in out_shapes],
        out_shape=[jax.ShapeDtypeStruct(s, _f32) for s in out_shapes],
    )(*ins)


def _build_conv_matrix(Wc, bc):
    o, c, di, dj, p, q = np.meshgrid(np.arange(3), np.arange(3), np.arange(3),
                                     np.arange(3), np.arange(6), np.arange(6),
                                     indexing="ij")
    k = (c * 64 + (p + di) * 8 + (q + dj)).ravel()
    m = (o * 36 + p * 6 + q).ravel()
    w = Wc[o.ravel(), c.ravel(), di.ravel(), dj.ravel()]
    M = jnp.zeros((192, 108), _f32).at[k, m].add(w)
    return M, jnp.repeat(bc, 36)[None, :]


def kernel(x, bd_pred, Wc, bc, Wlin, blin, Wlins, blins, Wl1, bl1, Wr1, Wl2,
           bl2, Wr2, Wl3, bl3, Wr3, ln1_g, ln1_b, ln2_g, ln2_b, mp1_W, mp1_b,
           mp2_W, mp2_b, edge_index):
    M, bcvec = _build_conv_matrix(Wc, bc)
    x2 = x.reshape(N, 192)
    pad = EPAD - E
    src2 = jnp.concatenate(
        [edge_index[0], jnp.zeros((pad,), jnp.int32)]).reshape(EROWS, B)
    dst2 = jnp.concatenate(
        [edge_index[1], jnp.full((pad,), DUMMY, jnp.int32)]).reshape(EROWS, B)

    row = lambda v: v[None, :]

    outd, dst0 = _sc_deg(src2, dst2)
    deg = (outd[:N] + outd[NA:NA + N]).reshape(N, 1)

    sx, h0, h1 = _tc_call(
        _tc_a_body,
        [x2, bd_pred, M, bcvec, Wlins.T, row(blins), Wlin.T, row(blin)],
        3, [(N, HID), (N, F2), (N, F2)])

    a0, a1 = _sc_agg(h0, h1, src2, dst0)
    a0, a1 = a0[:N], a1[:N]
    h0, h1, hr, dinv = _tc_call(
        _tc_1_body,
        [sx, a0, a1, deg, Wr1.T, row(bl1)],
        4, [(N, F2), (N, F2), (N, HID), (N, 1)])

    a0, a1 = _sc_agg(h0, h1, src2, dst2)
    a0, a1 = a0[:N], a1[:N]
    h0, h1, hr = _tc_call(
        _tc_mid_body,
        [a0, a1, dinv, hr, Wl1.T, row(ln1_g), row(ln1_b), Wr2.T, row(bl2)],
        3, [(N, F2), (N, F2), (N, HID)])

    a0, a1 = _sc_agg(h0, h1, src2, dst2)
    a0, a1 = a0[:N], a1[:N]
    h0, h1, hr = _tc_call(
        _tc_mid_body,
        [a0, a1, dinv, hr, Wl2.T, row(ln2_g), row(ln2_b), Wr3.T, row(bl3)],
        3, [(N, F2), (N, F2), (N, HID)])

    a0, a1 = _sc_agg(h0, h1, src2, dst2)
    a0, a1 = a0[:N], a1[:N]
    emb, lsm = _tc_call(
        _tc_4_body,
        [a0, a1, dinv, hr, Wl3.T, mp1_W.T, row(mp1_b), mp2_W.T, row(mp2_b)],
        2, [(N, HID), (N, 8)])

    return emb, lsm


# speedup vs baseline: 5.1723x; 1.1761x over previous
import functools

import numpy as np
import jax
import jax.numpy as jnp
from jax import lax
from jax.experimental import pallas as pl
from jax.experimental.pallas import tpu as pltpu
from jax.experimental.pallas import tpu_sc as plsc

N = 50000
E = 800000
HID = 64
F2 = 32
NB = 1000
NBLK = N // NB

B = 128
T = 2
RT = 400
NSC = RT // T
EROWS = 6528
EPAD = EROWS * B
DT = 4
DROWS = 204
NA = 50176
DUMMY = NA - 1
RPT = NA // 16
ZR = 196
OCH = 112

_f32 = jnp.float32


def _sc_agg_body(ht0, ht1, sd2, out0, out1, *scr):
    acc, zbuf = scr[0], scr[1]
    I = scr[2:10]
    g = scr[10:14]
    semI = scr[14:22]
    semG = scr[22:26]
    semS = scr[26:30]
    c = lax.axis_index("c")
    s = lax.axis_index("s")

    def _zb(i, car):
        zbuf[i, pl.ds(0, 16)] = jnp.zeros((16,), _f32)
        zbuf[i, pl.ds(16, 16)] = jnp.zeros((16,), _f32)
        return car
    lax.fori_loop(0, ZR, _zb, 0)

    def _zc(k, car):
        pltpu.sync_copy(zbuf, acc.at[pl.ds(s * RPT + k * ZR, ZR)])
        return car
    lax.fori_loop(0, RPT // ZR, _zc, 0)
    plsc.subcore_barrier()

    base = s * RT

    def _run(tab):
        def ifire(ch, q):
            pltpu.async_copy(sd2.at[base + ch], I[q], semI[q])

        def iwait(ch, q):
            pltpu.make_async_copy(sd2.at[base + ch], I[q], semI[q]).wait()

        def gfire(q, r):
            pltpu.async_copy(tab.at[I[q].at[0]], g[r], semG[r])

        def gwait(q, r):
            pltpu.make_async_copy(tab.at[I[q].at[0]], g[r], semG[r]).wait()

        def sfire(q, r):
            pltpu.async_copy(g[r], acc.at[I[q].at[1]], semS[r], add=True)

        def swait(q, r):
            pltpu.make_async_copy(g[r], acc.at[I[q].at[1]], semS[r]).wait()

        def slot(ch, j, first):
            if not first:
                swait((j - 2) % 8, (j - 2) % 4)
            iwait((j + 2) % 8, (j + 2) % 8)
            gfire((j + 2) % 8, (j + 2) % 4)
            gwait(j, j % 4)
            sfire(j, j % 4)
            ifire(ch + 6, (j + 6) % 8)

        for q in range(6):
            ifire(q, q)
        iwait(0, 0)
        gfire(0, 0)
        iwait(1, 1)
        gfire(1, 1)
        for j in range(8):
            slot(j, j, j < 2)

        def _body(k, car):
            c0 = 8 * k
            for j in range(8):
                slot(c0 + j, j, False)
            return car
        lax.fori_loop(1, RT // 8, _body, 0)

        swait(6, 2)
        swait(7, 3)
        gwait(0, 0)
        gwait(1, 1)
        for i in range(4):
            iwait(RT + 2 + i, 2 + i)

    @pl.when(c == 0)
    def _():
        _run(ht0)

    @pl.when(c == 1)
    def _():
        _run(ht1)

    plsc.subcore_barrier()

    def _out(dst):
        def _cp(k, car):
            r = s * RPT + k * OCH
            pltpu.sync_copy(acc.at[pl.ds(r, OCH)], g[0].at[pl.ds(0, OCH)])
            pltpu.sync_copy(g[0].at[pl.ds(0, OCH)], dst.at[pl.ds(r, OCH)])
            return car
        lax.fori_loop(0, RPT // OCH, _cp, 0)

    @pl.when(c == 0)
    def _():
        _out(out0)

    @pl.when(c == 1)
    def _():
        _out(out1)


def _sc_deg_body(sd2, outd, sd0, accd, zbufd, sdbuf, ones, sem):
    c = lax.axis_index("c")
    s = lax.axis_index("s")

    def _zb(i, car):
        zbufd[pl.ds(i * 16, 16)] = jnp.zeros((16,), _f32)
        return car
    lax.fori_loop(0, RPT // 16, _zb, 0)
    for t in range(8):
        ones[pl.ds(t * 16, 16)] = jnp.ones((16,), _f32)
    pltpu.sync_copy(zbufd, accd.at[pl.ds(s * RPT, RPT)])
    plsc.subcore_barrier()

    rbase = c * (EROWS // 2) + s * DROWS

    def _chunk(j, car):
        r0 = rbase + j * DT
        pltpu.sync_copy(sd2.at[pl.ds(r0, DT)], sdbuf)
        for t in range(DT):
            pltpu.async_copy(ones, accd.at[sdbuf.at[t, 1]], sem, add=True)
        for t in range(DT):
            pltpu.make_async_copy(ones, accd.at[sdbuf.at[t, 1]], sem).wait()
        for t in range(DT):
            def _cmp(i, car2):
                sv = sdbuf[t, 0, pl.ds(i * 16, 16)]
                dv = sdbuf[t, 1, pl.ds(i * 16, 16)]
                sdbuf[t, 1, pl.ds(i * 16, 16)] = jnp.where(
                    sv == dv, jnp.full((16,), DUMMY, jnp.int32), dv)
                return car2
            lax.fori_loop(0, B // 16, _cmp, 0)
        pltpu.sync_copy(sdbuf, sd0.at[pl.ds(r0, DT)])
        return car
    lax.fori_loop(0, DROWS // DT, _chunk, 0)
    plsc.subcore_barrier()
    pltpu.sync_copy(accd.at[pl.ds(s * RPT, RPT)], zbufd)
    pltpu.sync_copy(zbufd, outd.at[pl.ds(c * NA + s * RPT, RPT)])


_SC_MESH = plsc.VectorSubcoreMesh(core_axis_name="c", subcore_axis_name="s")

_sc_agg = pl.kernel(
    _sc_agg_body,
    out_type=[jax.ShapeDtypeStruct((NA, F2), _f32),
              jax.ShapeDtypeStruct((NA, F2), _f32)],
    mesh=_SC_MESH,
    compiler_params=pltpu.CompilerParams(use_tc_tiling_on_sc=False),
    scratch_types=(
        [pltpu.VMEM_SHARED((NA, F2), _f32), pltpu.VMEM((ZR, F2), _f32)]
        + [pltpu.VMEM((2, B), jnp.int32)] * 8
        + [pltpu.VMEM((B, F2), _f32)] * 4
        + [pltpu.SemaphoreType.DMA] * 16
    ),
)

_sc_deg = pl.kernel(
    _sc_deg_body,
    out_type=[jax.ShapeDtypeStruct((2 * NA,), _f32),
              jax.ShapeDtypeStruct((EROWS, 2, B), jnp.int32)],
    mesh=_SC_MESH,
    compiler_params=pltpu.CompilerParams(use_tc_tiling_on_sc=False),
    scratch_types=[
        pltpu.VMEM_SHARED((NA,), _f32),
        pltpu.VMEM((RPT,), _f32),
        pltpu.VMEM((DT, 2, B), jnp.int32),
        pltpu.VMEM((B,), _f32),
        pltpu.SemaphoreType.DMA,
    ],
)


def _dot(a, b):
    return jnp.dot(a, b, preferred_element_type=_f32)


def _tc_a_body(x2, bd, M, bc, wst, bs, wnt, bn, sx, h0, h1):
    y = _dot(x2[...], M[...]) + bc[...]
    r = jax.nn.relu(jnp.concatenate([y, bd[...]], axis=1))
    sx[...] = _dot(r, wst[...]) + bs[...]
    xnb = _dot(r, wnt[...]) + bn[...]
    h0[...] = xnb[:, :F2]
    h1[...] = xnb[:, F2:]


def _tc_1_body(sx, a0, a1, dg, wrt, bl, h0, h1, hr, dinv):
    h = sx[...] + jnp.concatenate([a0[...], a1[...]], axis=1)
    h0[...] = h[:, :F2]
    h1[...] = h[:, F2:]
    hr[...] = _dot(h, wrt[...]) + bl[...]
    dinv[...] = 1.0 / jnp.maximum(dg[...], 1.0)


def _tc_mid_body(a0, a1, dinv, hri, wlt, g, b, wrt, bl, h0, h1, hro):
    u = _dot(jnp.concatenate([a0[...], a1[...]], axis=1) * dinv[...],
             wlt[...]) + hri[...]
    v = jax.nn.relu(u)
    mu = jnp.mean(v, axis=1, keepdims=True)
    var = jnp.mean((v - mu) ** 2, axis=1, keepdims=True)
    h = (v - mu) * lax.rsqrt(var + 1e-5) * g[...] + b[...]
    h0[...] = h[:, :F2]
    h1[...] = h[:, F2:]
    hro[...] = _dot(h, wrt[...]) + bl[...]


def _tc_4_body(a0, a1, dinv, hri, wlt, w1, b1, w2, b2, emb, lsm):
    e = _dot(jnp.concatenate([a0[...], a1[...]], axis=1) * dinv[...],
             wlt[...]) + hri[...]
    emb[...] = e
    p = _dot(jax.nn.relu(e), w1[...]) + b1[...]
    q = _dot(p, w2[...]) + b2[...]
    m = jnp.max(q, axis=1, keepdims=True)
    lsm[...] = q - (jnp.log(jnp.sum(jnp.exp(q - m), axis=1, keepdims=True))
                    + m)


def _rows(shape):
    return pl.BlockSpec((NB,) + shape[1:], lambda i: (i,) + (0,) * (len(shape) - 1))


def _full(shape):
    return pl.BlockSpec(shape, lambda i: (0,) * len(shape))


def _tc_call(body, ins, n_out, out_shapes):
    specs = [_rows(a.shape) if a.shape[0] == N else _full(a.shape) for a in ins]
    return pl.pallas_call(
        body,
        grid=(NBLK,),
        in_specs=specs,
        out_specs=[_rows(s) for s in out_shapes],
        out_shape=[jax.ShapeDtypeStruct(s, _f32) for s in out_shapes],
    )(*ins)


def _build_conv_matrix(Wc, bc):
    o, c, di, dj, p, q = np.meshgrid(np.arange(3), np.arange(3), np.arange(3),
                                     np.arange(3), np.arange(6), np.arange(6),
                                     indexing="ij")
    k = (c * 64 + (p + di) * 8 + (q + dj)).ravel()
    m = (o * 36 + p * 6 + q).ravel()
    w = Wc[o.ravel(), c.ravel(), di.ravel(), dj.ravel()]
    M = jnp.zeros((192, 108), _f32).at[k, m].add(w)
    return M, jnp.repeat(bc, 36)[None, :]


def kernel(x, bd_pred, Wc, bc, Wlin, blin, Wlins, blins, Wl1, bl1, Wr1, Wl2,
           bl2, Wr2, Wl3, bl3, Wr3, ln1_g, ln1_b, ln2_g, ln2_b, mp1_W, mp1_b,
           mp2_W, mp2_b, edge_index):
    M, bcvec = _build_conv_matrix(Wc, bc)
    x2 = x.reshape(N, 192)
    pad = EPAD - E
    src2 = jnp.concatenate(
        [edge_index[0], jnp.zeros((pad,), jnp.int32)]).reshape(EROWS, 1, B)
    dst2 = jnp.concatenate(
        [edge_index[1], jnp.full((pad,), DUMMY, jnp.int32)]).reshape(EROWS, 1, B)
    sd2 = jnp.concatenate([src2, dst2], axis=1)

    row = lambda v: v[None, :]

    outd, sd0 = _sc_deg(sd2)
    deg = (outd[:N] + outd[NA:NA + N]).reshape(N, 1)

    sx, h0, h1 = _tc_call(
        _tc_a_body,
        [x2, bd_pred, M, bcvec, Wlins.T, row(blins), Wlin.T, row(blin)],
        3, [(N, HID), (N, F2), (N, F2)])

    a0, a1 = _sc_agg(h0, h1, sd0)
    a0, a1 = a0[:N], a1[:N]
    h0, h1, hr, dinv = _tc_call(
        _tc_1_body,
        [sx, a0, a1, deg, Wr1.T, row(bl1)],
        4, [(N, F2), (N, F2), (N, HID), (N, 1)])

    a0, a1 = _sc_agg(h0, h1, sd2)
    a0, a1 = a0[:N], a1[:N]
    h0, h1, hr = _tc_call(
        _tc_mid_body,
        [a0, a1, dinv, hr, Wl1.T, row(ln1_g), row(ln1_b), Wr2.T, row(bl2)],
        3, [(N, F2), (N, F2), (N, HID)])

    a0, a1 = _sc_agg(h0, h1, sd2)
    a0, a1 = a0[:N], a1[:N]
    h0, h1, hr = _tc_call(
        _tc_mid_body,
        [a0, a1, dinv, hr, Wl2.T, row(ln2_g), row(ln2_b), Wr3.T, row(bl3)],
        3, [(N, F2), (N, F2), (N, HID)])

    a0, a1 = _sc_agg(h0, h1, sd2)
    a0, a1 = a0[:N], a1[:N]
    emb, lsm = _tc_call(
        _tc_4_body,
        [a0, a1, dinv, hr, Wl3.T, mp1_W.T, row(mp1_b), mp2_W.T, row(mp2_b)],
        2, [(N, HID), (N, 8)])

    return emb, lsm


# speedup vs baseline: 5.4452x; 1.0528x over previous
import functools

import numpy as np
import jax
import jax.numpy as jnp
from jax import lax
from jax.experimental import pallas as pl
from jax.experimental.pallas import tpu as pltpu
from jax.experimental.pallas import tpu_sc as plsc

N = 50000
E = 800000
HID = 64
F2 = 32
NB = 1000
NBLK = N // NB

B = 128
T = 2
RT = 400
NSC = RT // T
EROWS = 6528
EPAD = EROWS * B
DT = 4
DROWS = 204
NA = 50176
DUMMY = NA - 1
RPT = NA // 16
ZR = 196
OCH = 112

_f32 = jnp.float32


def _sc_agg_body(ht0, ht1, sd2, out0, out1, *scr):
    acc, zbuf = scr[0], scr[1]
    I = scr[2:10]
    g = scr[10:14]
    semI = scr[14:22]
    semG = scr[22:26]
    semS = scr[26:30]
    c = lax.axis_index("c")
    s = lax.axis_index("s")

    def _zb(i, car):
        zbuf[i, pl.ds(0, 16)] = jnp.zeros((16,), _f32)
        zbuf[i, pl.ds(16, 16)] = jnp.zeros((16,), _f32)
        return car
    lax.fori_loop(0, ZR, _zb, 0)

    def _zc(k, car):
        pltpu.sync_copy(zbuf, acc.at[pl.ds(s * RPT + k * ZR, ZR)])
        return car
    lax.fori_loop(0, RPT // ZR, _zc, 0)
    plsc.subcore_barrier()

    base = s * RT

    def _run(tab):
        def ifire(ch, q):
            pltpu.async_copy(sd2.at[base + ch], I[q], semI[q])

        def iwait(ch, q):
            pltpu.make_async_copy(sd2.at[base + ch], I[q], semI[q]).wait()

        def gfire(q, r):
            pltpu.async_copy(tab.at[I[q].at[0]], g[r], semG[r])

        def gwait(q, r):
            pltpu.make_async_copy(tab.at[I[q].at[0]], g[r], semG[r]).wait()

        def sfire(q, r):
            pltpu.async_copy(g[r], acc.at[I[q].at[1]], semS[r], add=True)

        def swait(q, r):
            pltpu.make_async_copy(g[r], acc.at[I[q].at[1]], semS[r]).wait()

        def slot(ch, j, first):
            if not first:
                swait((j - 2) % 8, (j - 2) % 4)
            iwait((j + 2) % 8, (j + 2) % 8)
            gfire((j + 2) % 8, (j + 2) % 4)
            gwait(j, j % 4)
            sfire(j, j % 4)
            ifire(ch + 6, (j + 6) % 8)

        for q in range(6):
            ifire(q, q)
        iwait(0, 0)
        gfire(0, 0)
        iwait(1, 1)
        gfire(1, 1)
        for j in range(8):
            slot(j, j, j < 2)

        def _body(k, car):
            c0 = 8 * k
            for j in range(8):
                slot(c0 + j, j, False)
            return car
        lax.fori_loop(1, RT // 8, _body, 0)

        swait(6, 2)
        swait(7, 3)
        gwait(0, 0)
        gwait(1, 1)
        for i in range(4):
            iwait(RT + 2 + i, 2 + i)

    @pl.when(c == 0)
    def _():
        _run(ht0)

    @pl.when(c == 1)
    def _():
        _run(ht1)

    plsc.subcore_barrier()

    def _out(dst):
        def _cp(k, car):
            r = s * RPT + k * OCH
            pltpu.sync_copy(acc.at[pl.ds(r, OCH)], g[0].at[pl.ds(0, OCH)])
            pltpu.sync_copy(g[0].at[pl.ds(0, OCH)], dst.at[pl.ds(r, OCH)])
            return car
        lax.fori_loop(0, RPT // OCH, _cp, 0)

    @pl.when(c == 0)
    def _():
        _out(out0)

    @pl.when(c == 1)
    def _():
        _out(out1)


def _sc_deg_body(sd2, outd, sd0, accd, zbufd, sdbuf, ones, sem):
    c = lax.axis_index("c")
    s = lax.axis_index("s")

    def _zb(i, car):
        zbufd[pl.ds(i * 16, 16)] = jnp.zeros((16,), _f32)
        return car
    lax.fori_loop(0, RPT // 16, _zb, 0)
    for t in range(8):
        ones[pl.ds(t * 16, 16)] = jnp.ones((16,), _f32)
    pltpu.sync_copy(zbufd, accd.at[pl.ds(s * RPT, RPT)])
    plsc.subcore_barrier()

    rbase = c * (EROWS // 2) + s * DROWS

    def _chunk(j, car):
        r0 = rbase + j * DT
        pltpu.sync_copy(sd2.at[pl.ds(r0, DT)], sdbuf)
        for t in range(DT):
            pltpu.async_copy(ones, accd.at[sdbuf.at[t, 1]], sem, add=True)
        for t in range(DT):
            pltpu.make_async_copy(ones, accd.at[sdbuf.at[t, 1]], sem).wait()
        for t in range(DT):
            def _cmp(i, car2):
                sv = sdbuf[t, 0, pl.ds(i * 16, 16)]
                dv = sdbuf[t, 1, pl.ds(i * 16, 16)]
                sdbuf[t, 1, pl.ds(i * 16, 16)] = jnp.where(
                    sv == dv, jnp.full((16,), DUMMY, jnp.int32), dv)
                return car2
            lax.fori_loop(0, B // 16, _cmp, 0)
        pltpu.sync_copy(sdbuf, sd0.at[pl.ds(r0, DT)])
        return car
    lax.fori_loop(0, DROWS // DT, _chunk, 0)
    plsc.subcore_barrier()
    pltpu.sync_copy(accd.at[pl.ds(s * RPT, RPT)], zbufd)
    pltpu.sync_copy(zbufd, outd.at[pl.ds(c * NA + s * RPT, RPT)])


_SC_MESH = plsc.VectorSubcoreMesh(core_axis_name="c", subcore_axis_name="s")

_sc_agg = pl.kernel(
    _sc_agg_body,
    out_type=[jax.ShapeDtypeStruct((NA, F2), _f32),
              jax.ShapeDtypeStruct((NA, F2), _f32)],
    mesh=_SC_MESH,
    compiler_params=pltpu.CompilerParams(use_tc_tiling_on_sc=False),
    scratch_types=(
        [pltpu.VMEM_SHARED((NA, F2), _f32), pltpu.VMEM((ZR, F2), _f32)]
        + [pltpu.VMEM((2, B), jnp.int32)] * 8
        + [pltpu.VMEM((B, F2), _f32)] * 4
        + [pltpu.SemaphoreType.DMA] * 16
    ),
)

_sc_deg = pl.kernel(
    _sc_deg_body,
    out_type=[jax.ShapeDtypeStruct((2 * NA,), _f32),
              jax.ShapeDtypeStruct((EROWS, 2, B), jnp.int32)],
    mesh=_SC_MESH,
    compiler_params=pltpu.CompilerParams(use_tc_tiling_on_sc=False),
    scratch_types=[
        pltpu.VMEM_SHARED((NA,), _f32),
        pltpu.VMEM((RPT,), _f32),
        pltpu.VMEM((DT, 2, B), jnp.int32),
        pltpu.VMEM((B,), _f32),
        pltpu.SemaphoreType.DMA,
    ],
)


def _dot(a, b):
    return jnp.dot(a, b, preferred_element_type=_f32)


def _tc_a_body(x2, bd, M, bc, wst, bs, wnt, bn, sx, h0, h1):
    y = _dot(x2[...], M[...]) + bc[...]
    r = jax.nn.relu(jnp.concatenate([y, bd[...]], axis=1))
    sx[...] = _dot(r, wst[...]) + bs[...]
    xnb = _dot(r, wnt[...]) + bn[...]
    h0[...] = xnb[:, :F2]
    h1[...] = xnb[:, F2:]


def _tc_1_body(sx, a0, a1, dg, wrt, bl, h0, h1, hr, dinv):
    h = sx[...] + jnp.concatenate([a0[...], a1[...]], axis=1)
    h0[...] = h[:, :F2]
    h1[...] = h[:, F2:]
    hr[...] = _dot(h, wrt[...]) + bl[...]
    dinv[...] = 1.0 / jnp.maximum(dg[...], 1.0)


def _tc_mid_body(a0, a1, dinv, hri, wlt, g, b, wrt, bl, h0, h1, hro):
    u = _dot(jnp.concatenate([a0[...], a1[...]], axis=1) * dinv[...],
             wlt[...]) + hri[...]
    v = jax.nn.relu(u)
    mu = jnp.mean(v, axis=1, keepdims=True)
    var = jnp.mean((v - mu) ** 2, axis=1, keepdims=True)
    h = (v - mu) * lax.rsqrt(var + 1e-5) * g[...] + b[...]
    h0[...] = h[:, :F2]
    h1[...] = h[:, F2:]
    hro[...] = _dot(h, wrt[...]) + bl[...]


def _tc_4_body(a0, a1, dinv, hri, wlt, w1, b1, w2, b2, emb, lsm):
    e = _dot(jnp.concatenate([a0[...], a1[...]], axis=1) * dinv[...],
             wlt[...]) + hri[...]
    emb[...] = e
    p = _dot(jax.nn.relu(e), w1[...]) + b1[...]
    q = _dot(p, w2[...]) + b2[...]
    m = jnp.max(q, axis=1, keepdims=True)
    lsm[...] = q - (jnp.log(jnp.sum(jnp.exp(q - m), axis=1, keepdims=True))
                    + m)


def _rows(shape):
    return pl.BlockSpec((NB,) + shape[1:], lambda i: (i,) + (0,) * (len(shape) - 1))


def _full(shape):
    return pl.BlockSpec(shape, lambda i: (0,) * len(shape))


def _tc_call(body, ins, n_out, out_shapes):
    specs = [_rows(a.shape) if a.shape[0] in (N, NA) else _full(a.shape)
             for a in ins]
    return pl.pallas_call(
        body,
        grid=(NBLK,),
        in_specs=specs,
        out_specs=[_rows(s) for s in out_shapes],
        out_shape=[jax.ShapeDtypeStruct(s, _f32) for s in out_shapes],
    )(*ins)


def _build_conv_matrix(Wc, bc):
    o, c, di, dj, p, q = np.meshgrid(np.arange(3), np.arange(3), np.arange(3),
                                     np.arange(3), np.arange(6), np.arange(6),
                                     indexing="ij")
    k = (c * 64 + (p + di) * 8 + (q + dj)).ravel()
    m = (o * 36 + p * 6 + q).ravel()
    w = Wc[o.ravel(), c.ravel(), di.ravel(), dj.ravel()]
    M = jnp.zeros((192, 108), _f32).at[k, m].add(w)
    return M, jnp.repeat(bc, 36)[None, :]


def kernel(x, bd_pred, Wc, bc, Wlin, blin, Wlins, blins, Wl1, bl1, Wr1, Wl2,
           bl2, Wr2, Wl3, bl3, Wr3, ln1_g, ln1_b, ln2_g, ln2_b, mp1_W, mp1_b,
           mp2_W, mp2_b, edge_index):
    M, bcvec = _build_conv_matrix(Wc, bc)
    x2 = x.reshape(N, 192)
    pad = EPAD - E
    src2 = jnp.concatenate(
        [edge_index[0], jnp.zeros((pad,), jnp.int32)]).reshape(EROWS, 1, B)
    dst2 = jnp.concatenate(
        [edge_index[1], jnp.full((pad,), DUMMY, jnp.int32)]).reshape(EROWS, 1, B)
    sd2 = jnp.concatenate([src2, dst2], axis=1)

    row = lambda v: v[None, :]

    outd, sd0 = _sc_deg(sd2)
    deg = (outd[:N] + outd[NA:NA + N]).reshape(N, 1)

    sx, h0, h1 = _tc_call(
        _tc_a_body,
        [x2, bd_pred, M, bcvec, Wlins.T, row(blins), Wlin.T, row(blin)],
        3, [(N, HID), (N, F2), (N, F2)])

    a0, a1 = _sc_agg(h0, h1, sd0)
    h0, h1, hr, dinv = _tc_call(
        _tc_1_body,
        [sx, a0, a1, deg, Wr1.T, row(bl1)],
        4, [(N, F2), (N, F2), (N, HID), (N, 1)])

    a0, a1 = _sc_agg(h0, h1, sd2)
    h0, h1, hr = _tc_call(
        _tc_mid_body,
        [a0, a1, dinv, hr, Wl1.T, row(ln1_g), row(ln1_b), Wr2.T, row(bl2)],
        3, [(N, F2), (N, F2), (N, HID)])

    a0, a1 = _sc_agg(h0, h1, sd2)
    h0, h1, hr = _tc_call(
        _tc_mid_body,
        [a0, a1, dinv, hr, Wl2.T, row(ln2_g), row(ln2_b), Wr3.T, row(bl3)],
        3, [(N, F2), (N, F2), (N, HID)])

    a0, a1 = _sc_agg(h0, h1, sd2)
    emb, lsm = _tc_call(
        _tc_4_body,
        [a0, a1, dinv, hr, Wl3.T, mp1_W.T, row(mp1_b), mp2_W.T, row(mp2_b)],
        2, [(N, HID), (N, 8)])

    return emb, lsm
